# Initial kernel scaffold; baseline (speedup 1.0000x reference)
import jax
import jax.numpy as jnp
from jax.experimental import pallas as pl


def kernel(hidden_states, router_weight, router_bias, gate_up_proj, gate_up_proj_bias, down_proj, down_proj_bias):
    raise NotImplementedError("write your pallas kernel here")


# speedup vs baseline: 76.9179x; 76.9179x over previous
import functools

import jax
import jax.numpy as jnp
from jax import lax
from jax.experimental import pallas as pl
from jax.experimental.pallas import tpu as pltpu
from jax.experimental.pallas import tpu_sc as plsc

T = 2048
H = 1024
E = 64
I = 512
TWO_I = 2 * I
ALPHA, LIMIT = 1.702, 7.0

TM = 128
NT = 96
P_CAP = NT * TM

NC, NS = 2, 16
NW = NC * NS
TPW = T // NW
WREP = 128


def _router_body(x_ref, rw_ref, rb_ref, scores_ref, pos0_ref, pos1_ref,
                 w0_ref, w1_ref, te_ref, meta_ref):
    x = x_ref[...]
    logits = jnp.dot(x, rw_ref[...],
                     preferred_element_type=jnp.float32) + rb_ref[...]
    ei = lax.broadcasted_iota(jnp.int32, (T, E), 1)
    m1 = jnp.max(logits, axis=1, keepdims=True)
    i1 = jnp.min(jnp.where(logits == m1, ei, E), axis=1, keepdims=True)
    oh1 = ei == i1
    masked = jnp.where(oh1, -jnp.inf, logits)
    m2 = jnp.max(masked, axis=1, keepdims=True)
    i2 = jnp.min(jnp.where(masked == m2, ei, E), axis=1, keepdims=True)
    oh2 = ei == i2
    d = jnp.exp(m2 - m1)
    ssum = 1.0 + d
    w1 = 1.0 / ssum
    w2 = d / ssum
    oh1f = oh1.astype(jnp.float32)
    oh2f = oh2.astype(jnp.float32)
    scores_ref[...] = w1 * oh1f + w2 * oh2f

    a = oh1f + oh2f
    ri = lax.broadcasted_iota(jnp.int32, (T, T), 0)
    ci = lax.broadcasted_iota(jnp.int32, (T, T), 1)
    ltri = (ri > ci).astype(jnp.float32)
    cx = jnp.dot(ltri, a, preferred_element_type=jnp.float32)

    counts_i = jnp.sum(a, axis=0, keepdims=True).astype(jnp.int32)
    tiles_i = (counts_i + (TM - 1)) // TM
    e1 = lax.broadcasted_iota(jnp.int32, (E, E), 0)
    e2 = lax.broadcasted_iota(jnp.int32, (E, E), 1)
    u = (e1 < e2).astype(jnp.float32)
    base = jnp.dot(tiles_i.astype(jnp.float32), u,
                   preferred_element_type=jnp.float32)
    pad_off = base * TM
    p = cx + pad_off
    pos0_ref[...] = jnp.sum(p * oh1f, axis=1, keepdims=True).astype(jnp.int32)
    pos1_ref[...] = jnp.sum(p * oh2f, axis=1, keepdims=True).astype(jnp.int32)
    w0_ref[...] = jnp.broadcast_to(w1, (T, WREP))
    w1_ref[...] = jnp.broadcast_to(w2, (T, WREP))

    ends = (base + tiles_i.astype(jnp.float32)).astype(jnp.int32)
    ji = lax.broadcasted_iota(jnp.int32, (NT, E), 0)
    temap = jnp.sum((ji >= ends).astype(jnp.int32), axis=1, keepdims=True)
    e_row = lax.broadcasted_iota(jnp.int32, (1, E), 1)
    last_used = jnp.max(jnp.where(tiles_i > 0, e_row, 0))
    te_ref[...] = jnp.minimum(temap, last_used)
    meta_ref[...] = jnp.reshape(jnp.sum(tiles_i), (1, 1))


def _router(flat, rw, rb):
    return pl.pallas_call(
        _router_body,
        out_shape=[
            jax.ShapeDtypeStruct((T, E), jnp.float32),
            jax.ShapeDtypeStruct((T, 1), jnp.int32),
            jax.ShapeDtypeStruct((T, 1), jnp.int32),
            jax.ShapeDtypeStruct((T, WREP), jnp.float32),
            jax.ShapeDtypeStruct((T, WREP), jnp.float32),
            jax.ShapeDtypeStruct((NT, 1), jnp.int32),
            jax.ShapeDtypeStruct((1, 1), jnp.int32),
        ],
    )(flat, rw, rb.reshape(1, E))


def _scatter_body(x_hbm, pos0_hbm, pos1_hbm, w0_hbm, w1_hbm, xs_hbm, ws_hbm,
                  idx0, idx1, xbuf, wb0, wb1, s0, s1, s2, s3):
    wid = lax.axis_index("s") * NC + lax.axis_index("c")
    b = wid * TPW
    pltpu.sync_copy(pos0_hbm.at[pl.ds(b, TPW)], idx0)
    pltpu.sync_copy(pos1_hbm.at[pl.ds(b, TPW)], idx1)
    pltpu.sync_copy(x_hbm.at[pl.ds(b, TPW)], xbuf)
    pltpu.sync_copy(w0_hbm.at[pl.ds(b, TPW)], wb0)
    pltpu.sync_copy(w1_hbm.at[pl.ds(b, TPW)], wb1)
    c0 = pltpu.async_copy(xbuf, xs_hbm.at[idx0], s0)
    c1 = pltpu.async_copy(xbuf, xs_hbm.at[idx1], s1)
    c2 = pltpu.async_copy(wb0, ws_hbm.at[idx0], s2)
    c3 = pltpu.async_copy(wb1, ws_hbm.at[idx1], s3)
    c0.wait()
    c1.wait()
    c2.wait()
    c3.wait()


def _mlp_body(te_ref, meta_ref, xs_ref, ws_ref, gu_ref, gub_ref, dn_ref,
              dnb_ref, y_ref):
    j = pl.program_id(0)

    @pl.when(j < meta_ref[0])
    def _():
        x = xs_ref[...]
        h = jnp.dot(x, gu_ref[0], preferred_element_type=jnp.float32)
        h = h + gub_ref[0]
        gate = jnp.minimum(h, LIMIT)
        glu = gate / (1.0 + jnp.exp(-ALPHA * gate))
        up = jnp.clip(pltpu.roll(h, TWO_I - 1, 1), -LIMIT, LIMIT) + 1.0
        ci = lax.broadcasted_iota(jnp.int32, (TM, TWO_I), 1)
        act = jnp.where((ci % 2) == 0, up * glu, 0.0)
        s1 = lax.broadcasted_iota(jnp.int32, (TWO_I, I), 0)
        s2 = lax.broadcasted_iota(jnp.int32, (TWO_I, I), 1)
        sel = (s1 == 2 * s2).astype(jnp.float32)
        actc = jnp.dot(act, sel, preferred_element_type=jnp.float32)
        y = jnp.dot(actc, dn_ref[0], preferred_element_type=jnp.float32)
        y = y + dnb_ref[0]
        y_ref[...] = y * ws_ref[...][:, 0:1]


def _grouped_mlp(te_arr, meta, xs, ws, gu, gub, dn, dnb):
    grid_spec = pltpu.PrefetchScalarGridSpec(
        num_scalar_prefetch=2,
        grid=(NT,),
        in_specs=[
            pl.BlockSpec((TM, H), lambda j, te, meta: (jnp.minimum(j, meta[0] - 1), 0)),
            pl.BlockSpec((TM, WREP), lambda j, te, meta: (jnp.minimum(j, meta[0] - 1), 0)),
            pl.BlockSpec((1, H, TWO_I), lambda j, te, meta: (te[j], 0, 0)),
            pl.BlockSpec((1, 1, TWO_I), lambda j, te, meta: (te[j], 0, 0)),
            pl.BlockSpec((1, I, H), lambda j, te, meta: (te[j], 0, 0)),
            pl.BlockSpec((1, 1, H), lambda j, te, meta: (te[j], 0, 0)),
        ],
        out_specs=pl.BlockSpec((TM, H), lambda j, te, meta: (j, 0)),
    )
    return pl.pallas_call(
        _mlp_body,
        grid_spec=grid_spec,
        out_shape=jax.ShapeDtypeStruct((P_CAP, H), jnp.float32),
        compiler_params=pltpu.CompilerParams(
            dimension_semantics=("arbitrary",)),
    )(te_arr, meta, xs, ws, gu, gub, dn, dnb)


_CH = 32


def _combine_body(y_hbm, pos0_hbm, pos1_hbm, out_hbm, idx0, idx1, y0, y1,
                  s0, s1):
    wid = lax.axis_index("s") * NC + lax.axis_index("c")
    for sub in range(TPW // _CH):
        b = wid * TPW + sub * _CH
        pltpu.sync_copy(pos0_hbm.at[pl.ds(b, _CH)], idx0)
        pltpu.sync_copy(pos1_hbm.at[pl.ds(b, _CH)], idx1)
        c0 = pltpu.async_copy(y_hbm.at[idx0], y0, s0)
        c1 = pltpu.async_copy(y_hbm.at[idx1], y1, s1)
        c0.wait()
        c1.wait()

        def row(i, _):
            def col(c, _):
                sl = pl.ds(c * 16, 16)
                y0[i, sl] = y0[i, sl] + y1[i, sl]
                return 0
            return lax.fori_loop(0, H // 16, col, 0)

        lax.fori_loop(0, _CH, row, 0)
        pltpu.sync_copy(y0, out_hbm.at[pl.ds(b, _CH)])


@functools.lru_cache(maxsize=1)
def _sc_kernels():
    mesh = plsc.VectorSubcoreMesh(core_axis_name="c", subcore_axis_name="s")
    scatter = pl.kernel(
        _scatter_body,
        mesh=mesh,
        out_type=[
            jax.ShapeDtypeStruct((P_CAP, H), jnp.float32),
            jax.ShapeDtypeStruct((P_CAP, WREP), jnp.float32),
        ],
        scratch_types=[
            pltpu.VMEM((TPW,), jnp.int32),
            pltpu.VMEM((TPW,), jnp.int32),
            pltpu.VMEM((TPW, H), jnp.float32),
            pltpu.VMEM((TPW, WREP), jnp.float32),
            pltpu.VMEM((TPW, WREP), jnp.float32),
            pltpu.SemaphoreType.DMA,
            pltpu.SemaphoreType.DMA,
            pltpu.SemaphoreType.DMA,
            pltpu.SemaphoreType.DMA,
        ],
    )
    combine = pl.kernel(
        _combine_body,
        mesh=mesh,
        out_type=jax.ShapeDtypeStruct((T, H), jnp.float32),
        scratch_types=[
            pltpu.VMEM((_CH,), jnp.int32),
            pltpu.VMEM((_CH,), jnp.int32),
            pltpu.VMEM((_CH, H), jnp.float32),
            pltpu.VMEM((_CH, H), jnp.float32),
            pltpu.SemaphoreType.DMA,
            pltpu.SemaphoreType.DMA,
        ],
    )
    return scatter, combine


def kernel(hidden_states, router_weight, router_bias, gate_up_proj,
           gate_up_proj_bias, down_proj, down_proj_bias):
    b, s, h = hidden_states.shape
    flat = hidden_states.reshape(T, H)
    scores, pos0, pos1, w0, w1, te, meta = _router(flat, router_weight,
                                                   router_bias)
    pos0 = pos0.reshape(T)
    pos1 = pos1.reshape(T)
    _scatter, _combine = _sc_kernels()
    xs, ws = _scatter(flat, pos0, pos1, w0, w1)
    y = _grouped_mlp(te.reshape(NT), meta.reshape(1), xs, ws, gate_up_proj,
                     gate_up_proj_bias.reshape(E, 1, TWO_I), down_proj,
                     down_proj_bias.reshape(E, 1, H))
    out = _combine(y, pos0, pos1)
    return out.reshape(b, s, h), scores


# speedup vs baseline: 83.9295x; 1.0912x over previous
import functools

import jax
import jax.numpy as jnp
from jax import lax
from jax.experimental import pallas as pl
from jax.experimental.pallas import tpu as pltpu
from jax.experimental.pallas import tpu_sc as plsc

T = 2048
H = 1024
E = 64
I = 512
TWO_I = 2 * I
ALPHA, LIMIT = 1.702, 7.0

TM = 128
NT = 96
P_CAP = NT * TM

NC, NS = 2, 16
NW = NC * NS
TPW = T // NW
WREP = 128


def _router_body(x_ref, rw_ref, rb_ref, scores_ref, pos0_ref, pos1_ref,
                 w0_ref, w1_ref, te_ref, meta_ref):
    x = x_ref[...]
    logits = jnp.dot(x, rw_ref[...],
                     preferred_element_type=jnp.float32) + rb_ref[...]
    ei = lax.broadcasted_iota(jnp.int32, (T, E), 1)
    m1 = jnp.max(logits, axis=1, keepdims=True)
    i1 = jnp.min(jnp.where(logits == m1, ei, E), axis=1, keepdims=True)
    oh1 = ei == i1
    masked = jnp.where(oh1, -jnp.inf, logits)
    m2 = jnp.max(masked, axis=1, keepdims=True)
    i2 = jnp.min(jnp.where(masked == m2, ei, E), axis=1, keepdims=True)
    oh2 = ei == i2
    d = jnp.exp(m2 - m1)
    ssum = 1.0 + d
    w1 = 1.0 / ssum
    w2 = d / ssum
    oh1f = oh1.astype(jnp.float32)
    oh2f = oh2.astype(jnp.float32)
    scores_ref[...] = w1 * oh1f + w2 * oh2f

    a = oh1f + oh2f
    ri = lax.broadcasted_iota(jnp.int32, (T, T), 0)
    ci = lax.broadcasted_iota(jnp.int32, (T, T), 1)
    ltri = (ri > ci).astype(jnp.float32)
    cx = jnp.dot(ltri, a, preferred_element_type=jnp.float32)

    counts_i = jnp.sum(a, axis=0, keepdims=True).astype(jnp.int32)
    tiles_i = (counts_i + (TM - 1)) // TM
    e1 = lax.broadcasted_iota(jnp.int32, (E, E), 0)
    e2 = lax.broadcasted_iota(jnp.int32, (E, E), 1)
    u = (e1 < e2).astype(jnp.float32)
    base = jnp.dot(tiles_i.astype(jnp.float32), u,
                   preferred_element_type=jnp.float32)
    pad_off = base * TM
    p = cx + pad_off
    pos0_ref[...] = jnp.sum(p * oh1f, axis=1, keepdims=True).astype(jnp.int32)
    pos1_ref[...] = jnp.sum(p * oh2f, axis=1, keepdims=True).astype(jnp.int32)
    w0_ref[...] = jnp.broadcast_to(w1, (T, WREP))
    w1_ref[...] = jnp.broadcast_to(w2, (T, WREP))

    ends = (base + tiles_i.astype(jnp.float32)).astype(jnp.int32)
    ji = lax.broadcasted_iota(jnp.int32, (NT, E), 0)
    temap = jnp.sum((ji >= ends).astype(jnp.int32), axis=1, keepdims=True)
    e_row = lax.broadcasted_iota(jnp.int32, (1, E), 1)
    last_used = jnp.max(jnp.where(tiles_i > 0, e_row, 0))
    te_ref[...] = jnp.minimum(temap, last_used)
    meta_ref[...] = jnp.reshape(jnp.sum(tiles_i), (1, 1))


def _router(flat, rw, rb):
    return pl.pallas_call(
        _router_body,
        out_shape=[
            jax.ShapeDtypeStruct((T, E), jnp.float32),
            jax.ShapeDtypeStruct((T, 1), jnp.int32),
            jax.ShapeDtypeStruct((T, 1), jnp.int32),
            jax.ShapeDtypeStruct((T, WREP), jnp.float32),
            jax.ShapeDtypeStruct((T, WREP), jnp.float32),
            jax.ShapeDtypeStruct((NT, 1), jnp.int32),
            jax.ShapeDtypeStruct((1, 1), jnp.int32),
        ],
    )(flat, rw, rb.reshape(1, E))


def _scatter_body(x_hbm, pos0_hbm, pos1_hbm, w0_hbm, w1_hbm, xs_hbm, ws_hbm,
                  idx0, idx1, xbuf, wb0, wb1, s0, s1, s2, s3):
    wid = lax.axis_index("s") * NC + lax.axis_index("c")
    b = wid * TPW
    pltpu.sync_copy(pos0_hbm.at[pl.ds(b, TPW)], idx0)
    pltpu.sync_copy(pos1_hbm.at[pl.ds(b, TPW)], idx1)
    pltpu.sync_copy(x_hbm.at[pl.ds(b, TPW)], xbuf)
    pltpu.sync_copy(w0_hbm.at[pl.ds(b, TPW)], wb0)
    pltpu.sync_copy(w1_hbm.at[pl.ds(b, TPW)], wb1)
    c0 = pltpu.async_copy(xbuf, xs_hbm.at[idx0], s0)
    c1 = pltpu.async_copy(xbuf, xs_hbm.at[idx1], s1)
    c2 = pltpu.async_copy(wb0, ws_hbm.at[idx0], s2)
    c3 = pltpu.async_copy(wb1, ws_hbm.at[idx1], s3)
    c0.wait()
    c1.wait()
    c2.wait()
    c3.wait()


def _mlp_body(te_ref, meta_ref, xs_ref, ws_ref, gu_ref, gub_ref, dn_ref,
              dnb_ref, y_ref):
    j = pl.program_id(0)

    @pl.when(j < meta_ref[0])
    def _():
        x = xs_ref[...]
        h = jnp.dot(x, gu_ref[0], preferred_element_type=jnp.float32)
        h = h + gub_ref[0]
        gate = jnp.minimum(h, LIMIT)
        glu = gate / (1.0 + jnp.exp(-ALPHA * gate))
        up = jnp.clip(pltpu.roll(h, TWO_I - 1, 1), -LIMIT, LIMIT) + 1.0
        ci = lax.broadcasted_iota(jnp.int32, (TM, TWO_I), 1)
        act = jnp.where((ci % 2) == 0, up * glu, 0.0)
        s1 = lax.broadcasted_iota(jnp.int32, (TWO_I, I), 0)
        s2 = lax.broadcasted_iota(jnp.int32, (TWO_I, I), 1)
        sel = (s1 == 2 * s2).astype(jnp.float32)
        actc = jnp.dot(act, sel, preferred_element_type=jnp.float32)
        y = jnp.dot(actc, dn_ref[0], preferred_element_type=jnp.float32)
        y = y + dnb_ref[0]
        y_ref[...] = y * ws_ref[...][:, 0:1]


def _grouped_mlp(te_arr, meta, xs, ws, gu, gub, dn, dnb):
    grid_spec = pltpu.PrefetchScalarGridSpec(
        num_scalar_prefetch=2,
        grid=(NT,),
        in_specs=[
            pl.BlockSpec((TM, H), lambda j, te, meta: (jnp.minimum(j, meta[0] - 1), 0)),
            pl.BlockSpec((TM, WREP), lambda j, te, meta: (jnp.minimum(j, meta[0] - 1), 0)),
            pl.BlockSpec((1, H, TWO_I), lambda j, te, meta: (te[j], 0, 0)),
            pl.BlockSpec((1, 1, TWO_I), lambda j, te, meta: (te[j], 0, 0)),
            pl.BlockSpec((1, I, H), lambda j, te, meta: (te[j], 0, 0)),
            pl.BlockSpec((1, 1, H), lambda j, te, meta: (te[j], 0, 0)),
        ],
        out_specs=pl.BlockSpec((TM, H),
                               lambda j, te, meta: (jnp.minimum(j, meta[0] - 1), 0)),
    )
    return pl.pallas_call(
        _mlp_body,
        grid_spec=grid_spec,
        out_shape=jax.ShapeDtypeStruct((P_CAP, H), jnp.float32),
        compiler_params=pltpu.CompilerParams(
            dimension_semantics=("arbitrary",)),
    )(te_arr, meta, xs, ws, gu, gub, dn, dnb)


_CH = 32


def _combine_body(y_hbm, pos0_hbm, pos1_hbm, out_hbm, idx0, idx1, y0, y1,
                  s0, s1):
    wid = lax.axis_index("s") * NC + lax.axis_index("c")
    for sub in range(TPW // _CH):
        b = wid * TPW + sub * _CH
        pltpu.sync_copy(pos0_hbm.at[pl.ds(b, _CH)], idx0)
        pltpu.sync_copy(pos1_hbm.at[pl.ds(b, _CH)], idx1)
        c0 = pltpu.async_copy(y_hbm.at[idx0], y0, s0)
        c1 = pltpu.async_copy(y_hbm.at[idx1], y1, s1)
        c0.wait()
        c1.wait()

        def row(i, _):
            for c in range(H // 16):
                sl = pl.ds(c * 16, 16)
                y0[i, sl] = y0[i, sl] + y1[i, sl]
            return 0

        lax.fori_loop(0, _CH, row, 0)
        pltpu.sync_copy(y0, out_hbm.at[pl.ds(b, _CH)])


@functools.lru_cache(maxsize=1)
def _sc_kernels():
    mesh = plsc.VectorSubcoreMesh(core_axis_name="c", subcore_axis_name="s")
    scatter = pl.kernel(
        _scatter_body,
        mesh=mesh,
        out_type=[
            jax.ShapeDtypeStruct((P_CAP, H), jnp.float32),
            jax.ShapeDtypeStruct((P_CAP, WREP), jnp.float32),
        ],
        scratch_types=[
            pltpu.VMEM((TPW,), jnp.int32),
            pltpu.VMEM((TPW,), jnp.int32),
            pltpu.VMEM((TPW, H), jnp.float32),
            pltpu.VMEM((TPW, WREP), jnp.float32),
            pltpu.VMEM((TPW, WREP), jnp.float32),
            pltpu.SemaphoreType.DMA,
            pltpu.SemaphoreType.DMA,
            pltpu.SemaphoreType.DMA,
            pltpu.SemaphoreType.DMA,
        ],
    )
    combine = pl.kernel(
        _combine_body,
        mesh=mesh,
        out_type=jax.ShapeDtypeStruct((T, H), jnp.float32),
        scratch_types=[
            pltpu.VMEM((_CH,), jnp.int32),
            pltpu.VMEM((_CH,), jnp.int32),
            pltpu.VMEM((_CH, H), jnp.float32),
            pltpu.VMEM((_CH, H), jnp.float32),
            pltpu.SemaphoreType.DMA,
            pltpu.SemaphoreType.DMA,
        ],
    )
    return scatter, combine


def kernel(hidden_states, router_weight, router_bias, gate_up_proj,
           gate_up_proj_bias, down_proj, down_proj_bias):
    b, s, h = hidden_states.shape
    flat = hidden_states.reshape(T, H)
    scores, pos0, pos1, w0, w1, te, meta = _router(flat, router_weight,
                                                   router_bias)
    pos0 = pos0.reshape(T)
    pos1 = pos1.reshape(T)
    _scatter, _combine = _sc_kernels()
    xs, ws = _scatter(flat, pos0, pos1, w0, w1)
    y = _grouped_mlp(te.reshape(NT), meta.reshape(1), xs, ws, gate_up_proj,
                     gate_up_proj_bias.reshape(E, 1, TWO_I), down_proj,
                     down_proj_bias.reshape(E, 1, H))
    out = _combine(y, pos0, pos1)
    return out.reshape(b, s, h), scores
